# Initial kernel scaffold; baseline (speedup 1.0000x reference)
import jax
import jax.numpy as jnp
from jax.experimental import pallas as pl


def kernel(x, edge_index, W, b):
    raise NotImplementedError("write your pallas kernel here")


# speedup vs baseline: 6.4255x; 6.4255x over previous
import functools

import jax
import jax.numpy as jnp
from jax import lax
from jax.experimental import pallas as pl
from jax.experimental.pallas import tpu as pltpu
from jax.experimental.pallas import tpu_sc as plsc

N_NODES = 10000
N_EDGES = 320000
D_FEAT = 128
D_ONES = 16
D_AUG = D_FEAT + D_ONES

NC = 2
NS = 16
NW = NC * NS
EDGES_PER_W = N_EDGES // NW
CHUNK = 80
NCHUNK = EDGES_PER_W // CHUNK
ROWS_PER_TILE = N_NODES // NS


def _sc_body(src_hbm, dst_hbm, xa_hbm, za_hbm, out_hbm,
             src_v, dst_v, rows_v, a_sh, sem):
    c = lax.axis_index("c")
    s = lax.axis_index("s")
    wid = s * NC + c
    lo = s * ROWS_PER_TILE

    pltpu.sync_copy(za_hbm.at[pl.ds(lo, ROWS_PER_TILE)],
                    a_sh.at[pl.ds(lo, ROWS_PER_TILE)])
    pltpu.sync_copy(src_hbm.at[wid], src_v)
    pltpu.sync_copy(dst_hbm.at[wid], dst_v)
    plsc.subcore_barrier()

    def chunk_body(j, carry):
        pltpu.async_copy(xa_hbm.at[src_v.at[j]], rows_v, sem).wait()
        pltpu.sync_copy(rows_v, a_sh.at[dst_v.at[j]], add=True)
        return carry

    lax.fori_loop(0, NCHUNK, chunk_body, 0)
    plsc.subcore_barrier()

    pltpu.sync_copy(a_sh.at[pl.ds(lo, ROWS_PER_TILE)],
                    out_hbm.at[c].at[pl.ds(lo, ROWS_PER_TILE)])


@functools.partial(jax.jit, static_argnums=())
def _sc_aggregate(src3, dst3, x_aug, zeros_a):
    mesh = plsc.VectorSubcoreMesh(core_axis_name="c", subcore_axis_name="s",
                                  num_cores=NC, num_subcores=NS)
    return pl.kernel(
        _sc_body,
        out_type=jax.ShapeDtypeStruct((NC, N_NODES, D_AUG), jnp.float32),
        mesh=mesh,
        compiler_params=pltpu.CompilerParams(use_tc_tiling_on_sc=False),
        scratch_types=[
            pltpu.VMEM((NCHUNK, CHUNK), jnp.int32),
            pltpu.VMEM((NCHUNK, CHUNK), jnp.int32),
            pltpu.VMEM((CHUNK, D_AUG), jnp.float32),
            pltpu.VMEM_SHARED((N_NODES, D_AUG), jnp.float32),
            pltpu.SemaphoreType.DMA,
        ],
    )(src3, dst3, x_aug, zeros_a)


def _mm_body(parts_ref, w_ref, o_ref):
    a = parts_ref[0] + parts_ref[1]
    o_ref[...] = jnp.dot(a, w_ref[...], preferred_element_type=jnp.float32)


def _tc_finish(parts, w_aug):
    return pl.pallas_call(
        _mm_body,
        out_shape=jax.ShapeDtypeStruct((N_NODES, D_FEAT), jnp.float32),
    )(parts, w_aug)


def kernel(x, edge_index, W, b):
    src3 = edge_index[0].astype(jnp.int32).reshape(NW, NCHUNK, CHUNK)
    dst3 = edge_index[1].astype(jnp.int32).reshape(NW, NCHUNK, CHUNK)
    x_aug = jnp.concatenate(
        [x, jnp.ones((N_NODES, D_ONES), jnp.float32)], axis=1)
    zeros_a = jnp.zeros((N_NODES, D_AUG), jnp.float32)
    parts = _sc_aggregate(src3, dst3, x_aug, zeros_a)
    w_aug = jnp.concatenate(
        [W, jnp.broadcast_to(b[None, :] / D_ONES, (D_ONES, D_FEAT))], axis=0)
    return _tc_finish(parts, w_aug)


# speedup vs baseline: 7.8345x; 1.2193x over previous
import functools

import jax
import jax.numpy as jnp
from jax import lax
from jax.experimental import pallas as pl
from jax.experimental.pallas import tpu as pltpu
from jax.experimental.pallas import tpu_sc as plsc

N_NODES = 10000
N_EDGES = 320000
D_FEAT = 128
D_ONES = 16
D_AUG = D_FEAT + D_ONES

NC = 2
NS = 16
NW = NC * NS
EDGES_PER_W = N_EDGES // NW
CHUNK = 40
NCHUNK = EDGES_PER_W // CHUNK
ROWS_PER_TILE = N_NODES // NS


def _sc_body(src_hbm, dst_hbm, xa_hbm, za_hbm, out_hbm,
             src_v, dst_v, rows0, rows1, a_sh, sem0, sem1):
    c = lax.axis_index("c")
    s = lax.axis_index("s")
    wid = s * NC + c
    lo = s * ROWS_PER_TILE

    pltpu.sync_copy(za_hbm.at[pl.ds(lo, ROWS_PER_TILE)],
                    a_sh.at[pl.ds(lo, ROWS_PER_TILE)])
    pltpu.sync_copy(src_hbm.at[wid], src_v)
    pltpu.sync_copy(dst_hbm.at[wid], dst_v)
    plsc.subcore_barrier()

    def gather(j, rows, sem):
        return pltpu.async_copy(xa_hbm.at[src_v.at[j]], rows, sem)

    def drain(rows, sem):
        pltpu.make_async_copy(xa_hbm.at[src_v.at[0]], rows, sem).wait()

    def scatter(j, rows):
        pltpu.sync_copy(rows, a_sh.at[dst_v.at[j]], add=True)

    gather(0, rows0, sem0)

    def pair_body(i, carry):
        j = 2 * i
        gather(j + 1, rows1, sem1)
        drain(rows0, sem0)
        scatter(j, rows0)
        gather(j + 2, rows0, sem0)
        drain(rows1, sem1)
        scatter(j + 1, rows1)
        return carry

    lax.fori_loop(0, NCHUNK // 2 - 1, pair_body, 0)
    gather(NCHUNK - 1, rows1, sem1)
    drain(rows0, sem0)
    scatter(NCHUNK - 2, rows0)
    drain(rows1, sem1)
    scatter(NCHUNK - 1, rows1)
    plsc.subcore_barrier()

    pltpu.sync_copy(a_sh.at[pl.ds(lo, ROWS_PER_TILE)],
                    out_hbm.at[c].at[pl.ds(lo, ROWS_PER_TILE)])


@functools.partial(jax.jit, static_argnums=())
def _sc_aggregate(src3, dst3, x_aug, zeros_a):
    mesh = plsc.VectorSubcoreMesh(core_axis_name="c", subcore_axis_name="s",
                                  num_cores=NC, num_subcores=NS)
    return pl.kernel(
        _sc_body,
        out_type=jax.ShapeDtypeStruct((NC, N_NODES, D_AUG), jnp.float32),
        mesh=mesh,
        compiler_params=pltpu.CompilerParams(use_tc_tiling_on_sc=False),
        scratch_types=[
            pltpu.VMEM((NCHUNK, CHUNK), jnp.int32),
            pltpu.VMEM((NCHUNK, CHUNK), jnp.int32),
            pltpu.VMEM((CHUNK, D_AUG), jnp.float32),
            pltpu.VMEM((CHUNK, D_AUG), jnp.float32),
            pltpu.VMEM_SHARED((N_NODES, D_AUG), jnp.float32),
            pltpu.SemaphoreType.DMA,
            pltpu.SemaphoreType.DMA,
        ],
    )(src3, dst3, x_aug, zeros_a)


def _mm_body(parts_ref, w_ref, o_ref):
    a = parts_ref[0] + parts_ref[1]
    o_ref[...] = jnp.dot(a, w_ref[...], preferred_element_type=jnp.float32)


def _tc_finish(parts, w_aug):
    return pl.pallas_call(
        _mm_body,
        out_shape=jax.ShapeDtypeStruct((N_NODES, D_FEAT), jnp.float32),
    )(parts, w_aug)


def kernel(x, edge_index, W, b):
    src3 = edge_index[0].astype(jnp.int32).reshape(NW, NCHUNK, CHUNK)
    dst3 = edge_index[1].astype(jnp.int32).reshape(NW, NCHUNK, CHUNK)
    x_aug = jnp.concatenate(
        [x, jnp.ones((N_NODES, D_ONES), jnp.float32)], axis=1)
    zeros_a = jnp.zeros((N_NODES, D_AUG), jnp.float32)
    parts = _sc_aggregate(src3, dst3, x_aug, zeros_a)
    w_aug = jnp.concatenate(
        [W, jnp.broadcast_to(b[None, :] / D_ONES, (D_ONES, D_FEAT))], axis=0)
    return _tc_finish(parts, w_aug)


# speedup vs baseline: 8.7696x; 1.1194x over previous
import functools

import jax
import jax.numpy as jnp
from jax import lax
from jax.experimental import pallas as pl
from jax.experimental.pallas import tpu as pltpu
from jax.experimental.pallas import tpu_sc as plsc

N_NODES = 10000
N_EDGES = 320000
D_FEAT = 128
D_CNT = 16

NC = 2
NS = 16
NW = NC * NS
EDGES_PER_W = N_EDGES // NW
CHUNK = 40
NCHUNK = EDGES_PER_W // CHUNK
ROWS_PER_TILE = N_NODES // NS


def _sc_body(src_hbm, dst_hbm, x_hbm, za_hbm, zc_hbm, outa_hbm, outc_hbm,
             src_v, dst_v, rows0, rows1, ones_v, a_sh, c_sh, sem0, sem1):
    c = lax.axis_index("c")
    s = lax.axis_index("s")
    wid = s * NC + c
    lo = s * ROWS_PER_TILE

    pltpu.sync_copy(za_hbm.at[pl.ds(lo, ROWS_PER_TILE)],
                    a_sh.at[pl.ds(lo, ROWS_PER_TILE)])
    pltpu.sync_copy(zc_hbm.at[pl.ds(lo, ROWS_PER_TILE)],
                    c_sh.at[pl.ds(lo, ROWS_PER_TILE)])
    pltpu.sync_copy(src_hbm.at[wid], src_v)
    pltpu.sync_copy(dst_hbm.at[wid], dst_v)

    def fill_ones(i, carry):
        ones_v[i] = jnp.ones((D_CNT,), jnp.float32)
        return carry
    lax.fori_loop(0, CHUNK, fill_ones, 0)
    plsc.subcore_barrier()

    def gather(j, rows, sem):
        return pltpu.async_copy(x_hbm.at[src_v.at[j]], rows, sem)

    def drain(rows, sem):
        pltpu.make_async_copy(x_hbm.at[src_v.at[0]], rows, sem).wait()

    def scatter(j, rows):
        pltpu.sync_copy(rows, a_sh.at[dst_v.at[j]], add=True)
        pltpu.sync_copy(ones_v, c_sh.at[dst_v.at[j]], add=True)

    gather(0, rows0, sem0)

    def pair_body(i, carry):
        j = 2 * i
        gather(j + 1, rows1, sem1)
        drain(rows0, sem0)
        scatter(j, rows0)
        gather(j + 2, rows0, sem0)
        drain(rows1, sem1)
        scatter(j + 1, rows1)
        return carry

    lax.fori_loop(0, (NCHUNK - 1) // 2, pair_body, 0)
    if NCHUNK % 2 == 1:
        drain(rows0, sem0)
        scatter(NCHUNK - 1, rows0)
    else:
        gather(NCHUNK - 1, rows1, sem1)
        drain(rows0, sem0)
        scatter(NCHUNK - 2, rows0)
        drain(rows1, sem1)
        scatter(NCHUNK - 1, rows1)
    plsc.subcore_barrier()

    pltpu.sync_copy(a_sh.at[pl.ds(lo, ROWS_PER_TILE)],
                    outa_hbm.at[c].at[pl.ds(lo, ROWS_PER_TILE)])
    pltpu.sync_copy(c_sh.at[pl.ds(lo, ROWS_PER_TILE)],
                    outc_hbm.at[c].at[pl.ds(lo, ROWS_PER_TILE)])


def _sc_aggregate(src3, dst3, x, zeros_a, zeros_c):
    mesh = plsc.VectorSubcoreMesh(core_axis_name="c", subcore_axis_name="s",
                                  num_cores=NC, num_subcores=NS)
    return pl.kernel(
        _sc_body,
        out_type=(
            jax.ShapeDtypeStruct((NC, N_NODES, D_FEAT), jnp.float32),
            jax.ShapeDtypeStruct((NC, N_NODES, D_CNT), jnp.float32),
        ),
        mesh=mesh,
        compiler_params=pltpu.CompilerParams(use_tc_tiling_on_sc=False),
        scratch_types=[
            pltpu.VMEM((NCHUNK, CHUNK), jnp.int32),
            pltpu.VMEM((NCHUNK, CHUNK), jnp.int32),
            pltpu.VMEM((CHUNK, D_FEAT), jnp.float32),
            pltpu.VMEM((CHUNK, D_FEAT), jnp.float32),
            pltpu.VMEM((CHUNK, D_CNT), jnp.float32),
            pltpu.VMEM_SHARED((N_NODES, D_FEAT), jnp.float32),
            pltpu.VMEM_SHARED((N_NODES, D_CNT), jnp.float32),
            pltpu.SemaphoreType.DMA,
            pltpu.SemaphoreType.DMA,
        ],
    )(src3, dst3, x, zeros_a, zeros_c)


def _mm_body(a_ref, c_ref, w_ref, b_ref, o_ref):
    a = a_ref[0] + a_ref[1]
    cnt = c_ref[0, :, :1] + c_ref[1, :, :1]
    o_ref[...] = jnp.dot(a, w_ref[...],
                         preferred_element_type=jnp.float32) + cnt * b_ref[...]


def _tc_finish(parts_a, parts_c, W, b2d):
    return pl.pallas_call(
        _mm_body,
        out_shape=jax.ShapeDtypeStruct((N_NODES, D_FEAT), jnp.float32),
    )(parts_a, parts_c, W, b2d)


def kernel(x, edge_index, W, b):
    src3 = edge_index[0].astype(jnp.int32).reshape(NW, NCHUNK, CHUNK)
    dst3 = edge_index[1].astype(jnp.int32).reshape(NW, NCHUNK, CHUNK)
    zeros_a = jnp.zeros((N_NODES, D_FEAT), jnp.float32)
    zeros_c = jnp.zeros((N_NODES, D_CNT), jnp.float32)
    parts_a, parts_c = _sc_aggregate(src3, dst3, x, zeros_a, zeros_c)
    return _tc_finish(parts_a, parts_c, W, b.reshape(1, D_FEAT))


# speedup vs baseline: 10.6786x; 1.2177x over previous
import functools

import jax
import jax.numpy as jnp
from jax import lax
from jax.experimental import pallas as pl
from jax.experimental.pallas import tpu as pltpu
from jax.experimental.pallas import tpu_sc as plsc

N_NODES = 10000
N_EDGES = 320000
D_FEAT = 128
D_CNT = 16

NC = 2
NS = 16
NW = NC * NS
EDGES_PER_W = N_EDGES // NW
CHUNK = 80
NCHUNK = EDGES_PER_W // CHUNK
NBLK = 5
BLKCH = NCHUNK // NBLK
ROWS_PER_TILE = N_NODES // NS


def _sc_body(src_hbm, dst_hbm, x_hbm, za_hbm, zc_hbm, outa_hbm, outc_hbm,
             src_a, src_b, dst_a, dst_b, rows0, rows1, ones_v,
             a_sh, c_sh, sem0, sem1, semi):
    c = lax.axis_index("c")
    s = lax.axis_index("s")
    wid = s * NC + c
    lo = s * ROWS_PER_TILE

    pltpu.sync_copy(za_hbm.at[pl.ds(lo, ROWS_PER_TILE)],
                    a_sh.at[pl.ds(lo, ROWS_PER_TILE)])
    pltpu.sync_copy(zc_hbm.at[pl.ds(lo, ROWS_PER_TILE)],
                    c_sh.at[pl.ds(lo, ROWS_PER_TILE)])
    pltpu.sync_copy(src_hbm.at[wid].at[pl.ds(0, BLKCH)], src_a)
    pltpu.sync_copy(dst_hbm.at[wid].at[pl.ds(0, BLKCH)], dst_a)

    def fill_ones(i, carry):
        ones_v[i] = jnp.ones((D_CNT,), jnp.float32)
        return carry
    lax.fori_loop(0, CHUNK, fill_ones, 0)
    plsc.subcore_barrier()

    def gather(sv, j, rows, sem):
        return pltpu.async_copy(x_hbm.at[sv.at[j]], rows, sem)

    def drain(rows, sem):
        pltpu.make_async_copy(x_hbm.at[src_a.at[0]], rows, sem).wait()

    def scatter(dv, j, rows):
        pltpu.sync_copy(rows, a_sh.at[dv.at[j]], add=True)
        pltpu.sync_copy(ones_v, c_sh.at[dv.at[j]], add=True)

    idx_bufs = (src_a, dst_a), (src_b, dst_b)
    for blk in range(NBLK):
        sv, dv = idx_bufs[blk % 2]
        nsv, ndv = idx_bufs[(blk + 1) % 2]
        if blk + 1 < NBLK:
            pltpu.async_copy(
                src_hbm.at[wid].at[pl.ds((blk + 1) * BLKCH, BLKCH)], nsv, semi)
            pltpu.async_copy(
                dst_hbm.at[wid].at[pl.ds((blk + 1) * BLKCH, BLKCH)], ndv, semi)

        gather(sv, 0, rows0, sem0)

        def pair_body(i, carry, sv=sv, dv=dv):
            j = 2 * i
            gather(sv, j + 1, rows1, sem1)
            drain(rows0, sem0)
            scatter(dv, j, rows0)
            gather(sv, j + 2, rows0, sem0)
            drain(rows1, sem1)
            scatter(dv, j + 1, rows1)
            return carry

        lax.fori_loop(0, (BLKCH - 1) // 2, pair_body, 0)
        drain(rows0, sem0)
        scatter(dv, BLKCH - 1, rows0)

        if blk + 1 < NBLK:
            pltpu.make_async_copy(
                src_hbm.at[wid].at[pl.ds(0, BLKCH)], nsv, semi).wait()
            pltpu.make_async_copy(
                dst_hbm.at[wid].at[pl.ds(0, BLKCH)], ndv, semi).wait()
    plsc.subcore_barrier()

    pltpu.sync_copy(a_sh.at[pl.ds(lo, ROWS_PER_TILE)],
                    outa_hbm.at[c].at[pl.ds(lo, ROWS_PER_TILE)])
    pltpu.sync_copy(c_sh.at[pl.ds(lo, ROWS_PER_TILE)],
                    outc_hbm.at[c].at[pl.ds(lo, ROWS_PER_TILE)])


def _sc_aggregate(src3, dst3, x, zeros_a, zeros_c):
    mesh = plsc.VectorSubcoreMesh(core_axis_name="c", subcore_axis_name="s",
                                  num_cores=NC, num_subcores=NS)
    return pl.kernel(
        _sc_body,
        out_type=(
            jax.ShapeDtypeStruct((NC, N_NODES, D_FEAT), jnp.float32),
            jax.ShapeDtypeStruct((NC, N_NODES, D_CNT), jnp.float32),
        ),
        mesh=mesh,
        compiler_params=pltpu.CompilerParams(use_tc_tiling_on_sc=False),
        scratch_types=[
            pltpu.VMEM((BLKCH, CHUNK), jnp.int32),
            pltpu.VMEM((BLKCH, CHUNK), jnp.int32),
            pltpu.VMEM((BLKCH, CHUNK), jnp.int32),
            pltpu.VMEM((BLKCH, CHUNK), jnp.int32),
            pltpu.VMEM((CHUNK, D_FEAT), jnp.float32),
            pltpu.VMEM((CHUNK, D_FEAT), jnp.float32),
            pltpu.VMEM((CHUNK, D_CNT), jnp.float32),
            pltpu.VMEM_SHARED((N_NODES, D_FEAT), jnp.float32),
            pltpu.VMEM_SHARED((N_NODES, D_CNT), jnp.float32),
            pltpu.SemaphoreType.DMA,
            pltpu.SemaphoreType.DMA,
            pltpu.SemaphoreType.DMA,
        ],
    )(src3, dst3, x, zeros_a, zeros_c)


def _mm_body(a_ref, c_ref, w_ref, b_ref, o_ref):
    a = a_ref[0] + a_ref[1]
    cnt = c_ref[0, :, :1] + c_ref[1, :, :1]
    o_ref[...] = jnp.dot(a, w_ref[...],
                         preferred_element_type=jnp.float32) + cnt * b_ref[...]


def _tc_finish(parts_a, parts_c, W, b2d):
    return pl.pallas_call(
        _mm_body,
        out_shape=jax.ShapeDtypeStruct((N_NODES, D_FEAT), jnp.float32),
    )(parts_a, parts_c, W, b2d)


def kernel(x, edge_index, W, b):
    src3 = edge_index[0].astype(jnp.int32).reshape(NW, NCHUNK, CHUNK)
    dst3 = edge_index[1].astype(jnp.int32).reshape(NW, NCHUNK, CHUNK)
    zeros_a = jnp.zeros((N_NODES, D_FEAT), jnp.float32)
    zeros_c = jnp.zeros((N_NODES, D_CNT), jnp.float32)
    parts_a, parts_c = _sc_aggregate(src3, dst3, x, zeros_a, zeros_c)
    return _tc_finish(parts_a, parts_c, W, b.reshape(1, D_FEAT))


# speedup vs baseline: 10.7604x; 1.0077x over previous
import functools

import jax
import jax.numpy as jnp
from jax import lax
from jax.experimental import pallas as pl
from jax.experimental.pallas import tpu as pltpu
from jax.experimental.pallas import tpu_sc as plsc

N_NODES = 10000
N_EDGES = 320000
D_FEAT = 128
D_CNT = 16

NC = 2
NS = 16
NW = NC * NS
EDGES_PER_W = N_EDGES // NW
CHUNK = 128
EDGES_PAD_W = 10240
PAD_PER_W = EDGES_PAD_W - EDGES_PER_W
NCHUNK = EDGES_PAD_W // CHUNK
NBLK = 8
BLKCH = NCHUNK // NBLK
ROWS_PER_TILE = N_NODES // NS
N_ACC = N_NODES + NS


def _sc_body(src_hbm, dst_hbm, x_hbm, za_hbm, zc_hbm, outa_hbm, outc_hbm,
             src_a, src_b, dst_a, dst_b, rows0, rows1, ones_v,
             a_sh, c_sh, sem0, sem1, semi):
    c = lax.axis_index("c")
    s = lax.axis_index("s")
    wid = s * NC + c
    lo = s * ROWS_PER_TILE

    pltpu.sync_copy(za_hbm.at[pl.ds(lo, ROWS_PER_TILE)],
                    a_sh.at[pl.ds(lo, ROWS_PER_TILE)])
    pltpu.sync_copy(zc_hbm.at[pl.ds(lo, ROWS_PER_TILE)],
                    c_sh.at[pl.ds(lo, ROWS_PER_TILE)])
    pltpu.sync_copy(src_hbm.at[wid].at[pl.ds(0, BLKCH)], src_a)
    pltpu.sync_copy(dst_hbm.at[wid].at[pl.ds(0, BLKCH)], dst_a)

    def fill_ones(i, carry):
        ones_v[i] = jnp.ones((D_CNT,), jnp.float32)
        return carry
    lax.fori_loop(0, CHUNK, fill_ones, 0)
    plsc.subcore_barrier()

    def gather(sv, j, rows, sem):
        return pltpu.async_copy(x_hbm.at[sv.at[j]], rows, sem)

    def drain(rows, sem):
        pltpu.make_async_copy(x_hbm.at[src_a.at[0]], rows, sem).wait()

    def scatter(dv, j, rows):
        pltpu.sync_copy(rows, a_sh.at[dv.at[j]], add=True)
        pltpu.sync_copy(ones_v, c_sh.at[dv.at[j]], add=True)

    idx_bufs = (src_a, dst_a), (src_b, dst_b)
    for blk in range(NBLK):
        sv, dv = idx_bufs[blk % 2]
        nsv, ndv = idx_bufs[(blk + 1) % 2]
        if blk + 1 < NBLK:
            pltpu.async_copy(
                src_hbm.at[wid].at[pl.ds((blk + 1) * BLKCH, BLKCH)], nsv, semi)
            pltpu.async_copy(
                dst_hbm.at[wid].at[pl.ds((blk + 1) * BLKCH, BLKCH)], ndv, semi)

        gather(sv, 0, rows0, sem0)

        def pair_body(i, carry, sv=sv, dv=dv):
            j = 2 * i
            gather(sv, j + 1, rows1, sem1)
            drain(rows0, sem0)
            scatter(dv, j, rows0)
            gather(sv, j + 2, rows0, sem0)
            drain(rows1, sem1)
            scatter(dv, j + 1, rows1)
            return carry

        lax.fori_loop(0, BLKCH // 2 - 1, pair_body, 0)
        gather(sv, BLKCH - 1, rows1, sem1)
        drain(rows0, sem0)
        scatter(dv, BLKCH - 2, rows0)
        drain(rows1, sem1)
        scatter(dv, BLKCH - 1, rows1)

        if blk + 1 < NBLK:
            pltpu.make_async_copy(
                src_hbm.at[wid].at[pl.ds(0, BLKCH)], nsv, semi).wait()
            pltpu.make_async_copy(
                dst_hbm.at[wid].at[pl.ds(0, BLKCH)], ndv, semi).wait()
    plsc.subcore_barrier()

    pltpu.sync_copy(a_sh.at[pl.ds(lo, ROWS_PER_TILE)],
                    outa_hbm.at[c].at[pl.ds(lo, ROWS_PER_TILE)])
    pltpu.sync_copy(c_sh.at[pl.ds(lo, ROWS_PER_TILE)],
                    outc_hbm.at[c].at[pl.ds(lo, ROWS_PER_TILE)])


def _sc_aggregate(src3, dst3, x, zeros_a, zeros_c):
    mesh = plsc.VectorSubcoreMesh(core_axis_name="c", subcore_axis_name="s",
                                  num_cores=NC, num_subcores=NS)
    return pl.kernel(
        _sc_body,
        out_type=(
            jax.ShapeDtypeStruct((NC, N_NODES, D_FEAT), jnp.float32),
            jax.ShapeDtypeStruct((NC, N_NODES, D_CNT), jnp.float32),
        ),
        mesh=mesh,
        compiler_params=pltpu.CompilerParams(use_tc_tiling_on_sc=False),
        scratch_types=[
            pltpu.VMEM((BLKCH, CHUNK), jnp.int32),
            pltpu.VMEM((BLKCH, CHUNK), jnp.int32),
            pltpu.VMEM((BLKCH, CHUNK), jnp.int32),
            pltpu.VMEM((BLKCH, CHUNK), jnp.int32),
            pltpu.VMEM((CHUNK, D_FEAT), jnp.float32),
            pltpu.VMEM((CHUNK, D_FEAT), jnp.float32),
            pltpu.VMEM((CHUNK, D_CNT), jnp.float32),
            pltpu.VMEM_SHARED((N_ACC, D_FEAT), jnp.float32),
            pltpu.VMEM_SHARED((N_ACC, D_CNT), jnp.float32),
            pltpu.SemaphoreType.DMA,
            pltpu.SemaphoreType.DMA,
            pltpu.SemaphoreType.DMA,
        ],
    )(src3, dst3, x, zeros_a, zeros_c)


def _mm_body(a_ref, c_ref, w_ref, b_ref, o_ref):
    a = a_ref[0] + a_ref[1]
    cnt = c_ref[0, :, :1] + c_ref[1, :, :1]
    o_ref[...] = jnp.dot(a, w_ref[...],
                         preferred_element_type=jnp.float32) + cnt * b_ref[...]


def _tc_finish(parts_a, parts_c, W, b2d):
    return pl.pallas_call(
        _mm_body,
        out_shape=jax.ShapeDtypeStruct((N_NODES, D_FEAT), jnp.float32),
    )(parts_a, parts_c, W, b2d)


def kernel(x, edge_index, W, b):
    src2 = edge_index[0].astype(jnp.int32).reshape(NW, EDGES_PER_W)
    dst2 = edge_index[1].astype(jnp.int32).reshape(NW, EDGES_PER_W)
    wids = jnp.arange(NW, dtype=jnp.int32)[:, None]
    k = jnp.arange(PAD_PER_W, dtype=jnp.int32)[None, :]
    src_pad = (wids * 313 + k * 97) % N_NODES
    dst_pad = jnp.broadcast_to(N_NODES + wids // NC, (NW, PAD_PER_W))
    src3 = jnp.concatenate([src2, src_pad], 1).reshape(NW, NCHUNK, CHUNK)
    dst3 = jnp.concatenate([dst2, dst_pad], 1).reshape(NW, NCHUNK, CHUNK)
    zeros_a = jnp.zeros((N_NODES, D_FEAT), jnp.float32)
    zeros_c = jnp.zeros((N_NODES, D_CNT), jnp.float32)
    parts_a, parts_c = _sc_aggregate(src3, dst3, x, zeros_a, zeros_c)
    return _tc_finish(parts_a, parts_c, W, b.reshape(1, D_FEAT))


# speedup vs baseline: 11.6235x; 1.0802x over previous
import functools

import jax
import jax.numpy as jnp
from jax import lax
from jax.experimental import pallas as pl
from jax.experimental.pallas import tpu as pltpu
from jax.experimental.pallas import tpu_sc as plsc

N_NODES = 10000
N_EDGES = 320000
D_FEAT = 128
D_CNT = 16

NC = 2
NS = 16
NW = NC * NS
EDGES_PER_W = N_EDGES // NW
CHUNK = 128
EDGES_PAD_W = 10240
PAD_PER_W = EDGES_PAD_W - EDGES_PER_W
NCHUNK = EDGES_PAD_W // CHUNK
NBLK = 8
BLKCH = NCHUNK // NBLK
ROWS_PER_TILE = N_NODES // NS
N_ACC = N_NODES + NS


def _sc_body(src_hbm, dst_hbm, x_hbm, za_hbm, zc_hbm, outa_hbm, outc_hbm,
             src_a, src_b, dst_a, dst_b, rows0, rows1, c_tile,
             a_sh, sem0, sem1, semi):
    c = lax.axis_index("c")
    s = lax.axis_index("s")
    wid = s * NC + c
    lo = s * ROWS_PER_TILE

    pltpu.sync_copy(za_hbm.at[pl.ds(lo, ROWS_PER_TILE)],
                    a_sh.at[pl.ds(lo, ROWS_PER_TILE)])
    pltpu.sync_copy(zc_hbm, c_tile)
    pltpu.sync_copy(src_hbm.at[wid].at[pl.ds(0, BLKCH)], src_a)
    pltpu.sync_copy(dst_hbm.at[wid].at[pl.ds(0, BLKCH)], dst_a)
    plsc.subcore_barrier()

    ones16 = jnp.ones((16,), jnp.float32)

    def gather(sv, j, rows, sem):
        return pltpu.async_copy(x_hbm.at[sv.at[j]], rows, sem)

    def drain(rows, sem):
        pltpu.make_async_copy(x_hbm.at[src_a.at[0]], rows, sem).wait()

    def counts(dv, j):
        for k in range(CHUNK // 16):
            idx = dv[j, pl.ds(k * 16, 16)]
            plsc.addupdate_scatter(c_tile, [idx], ones16)

    def scatter(dv, j, rows):
        pltpu.sync_copy(rows, a_sh.at[dv.at[j]], add=True)

    idx_bufs = (src_a, dst_a), (src_b, dst_b)
    for blk in range(NBLK):
        sv, dv = idx_bufs[blk % 2]
        nsv, ndv = idx_bufs[(blk + 1) % 2]
        if blk + 1 < NBLK:
            pltpu.async_copy(
                src_hbm.at[wid].at[pl.ds((blk + 1) * BLKCH, BLKCH)], nsv, semi)
            pltpu.async_copy(
                dst_hbm.at[wid].at[pl.ds((blk + 1) * BLKCH, BLKCH)], ndv, semi)

        gather(sv, 0, rows0, sem0)

        def pair_body(i, carry, sv=sv, dv=dv):
            j = 2 * i
            gather(sv, j + 1, rows1, sem1)
            counts(dv, j)
            drain(rows0, sem0)
            scatter(dv, j, rows0)
            gather(sv, j + 2, rows0, sem0)
            counts(dv, j + 1)
            drain(rows1, sem1)
            scatter(dv, j + 1, rows1)
            return carry

        lax.fori_loop(0, BLKCH // 2 - 1, pair_body, 0)
        gather(sv, BLKCH - 1, rows1, sem1)
        counts(dv, BLKCH - 2)
        drain(rows0, sem0)
        scatter(dv, BLKCH - 2, rows0)
        counts(dv, BLKCH - 1)
        drain(rows1, sem1)
        scatter(dv, BLKCH - 1, rows1)

        if blk + 1 < NBLK:
            pltpu.make_async_copy(
                src_hbm.at[wid].at[pl.ds(0, BLKCH)], nsv, semi).wait()
            pltpu.make_async_copy(
                dst_hbm.at[wid].at[pl.ds(0, BLKCH)], ndv, semi).wait()
    plsc.subcore_barrier()

    pltpu.sync_copy(a_sh.at[pl.ds(lo, ROWS_PER_TILE)],
                    outa_hbm.at[c].at[pl.ds(lo, ROWS_PER_TILE)])
    pltpu.sync_copy(c_tile.at[pl.ds(0, N_NODES)], outc_hbm.at[c].at[s])


def _sc_aggregate(src3, dst3, x, zeros_a, zeros_c):
    mesh = plsc.VectorSubcoreMesh(core_axis_name="c", subcore_axis_name="s",
                                  num_cores=NC, num_subcores=NS)
    return pl.kernel(
        _sc_body,
        out_type=(
            jax.ShapeDtypeStruct((NC, N_NODES, D_FEAT), jnp.float32),
            jax.ShapeDtypeStruct((NC, NS, N_NODES), jnp.float32),
        ),
        mesh=mesh,
        compiler_params=pltpu.CompilerParams(use_tc_tiling_on_sc=False,
                                             needs_layout_passes=False),
        scratch_types=[
            pltpu.VMEM((BLKCH, CHUNK), jnp.int32),
            pltpu.VMEM((BLKCH, CHUNK), jnp.int32),
            pltpu.VMEM((BLKCH, CHUNK), jnp.int32),
            pltpu.VMEM((BLKCH, CHUNK), jnp.int32),
            pltpu.VMEM((CHUNK, D_FEAT), jnp.float32),
            pltpu.VMEM((CHUNK, D_FEAT), jnp.float32),
            pltpu.VMEM((N_ACC,), jnp.float32),
            pltpu.VMEM_SHARED((N_ACC, D_FEAT), jnp.float32),
            pltpu.SemaphoreType.DMA,
            pltpu.SemaphoreType.DMA,
            pltpu.SemaphoreType.DMA,
        ],
    )(src3, dst3, x, zeros_a, zeros_c)


def _mm_body(a_ref, c_ref, w_ref, b_ref, o_ref):
    a = a_ref[0] + a_ref[1]
    cnt = jnp.sum(c_ref[...].reshape(NC * NS, N_NODES), axis=0)
    o_ref[...] = (jnp.dot(a, w_ref[...], preferred_element_type=jnp.float32)
                  + cnt[:, None] * b_ref[...])


def _tc_finish(parts_a, parts_c, W, b2d):
    return pl.pallas_call(
        _mm_body,
        out_shape=jax.ShapeDtypeStruct((N_NODES, D_FEAT), jnp.float32),
    )(parts_a, parts_c, W, b2d)


def kernel(x, edge_index, W, b):
    src2 = edge_index[0].astype(jnp.int32).reshape(NW, EDGES_PER_W)
    dst2 = edge_index[1].astype(jnp.int32).reshape(NW, EDGES_PER_W)
    wids = jnp.arange(NW, dtype=jnp.int32)[:, None]
    k = jnp.arange(PAD_PER_W, dtype=jnp.int32)[None, :]
    src_pad = (wids * 313 + k * 97) % N_NODES
    dst_pad = jnp.broadcast_to(N_NODES + wids // NC, (NW, PAD_PER_W))
    src3 = jnp.concatenate([src2, src_pad], 1).reshape(NW, NCHUNK, CHUNK)
    dst3 = jnp.concatenate([dst2, dst_pad], 1).reshape(NW, NCHUNK, CHUNK)
    zeros_a = jnp.zeros((N_NODES, D_FEAT), jnp.float32)
    zeros_c = jnp.zeros((N_ACC,), jnp.float32)
    parts_a, parts_c = _sc_aggregate(src3, dst3, x, zeros_a, zeros_c)
    return _tc_finish(parts_a, parts_c, W, b.reshape(1, D_FEAT))


# speedup vs baseline: 12.0339x; 1.0353x over previous
import jax
import jax.numpy as jnp
from jax import lax
from jax.experimental import pallas as pl
from jax.experimental.pallas import tpu as pltpu
from jax.experimental.pallas import tpu_sc as plsc

N_NODES = 10000
N_EDGES = 320000
D_FEAT = 128

NC = 2
NS = 16
NW = NC * NS
EDGES_PER_W = N_EDGES // NW
CHUNK = 128
NCHUNK = EDGES_PER_W // CHUNK
TAIL = EDGES_PER_W - NCHUNK * CHUNK
NBLK = 13
BLKCH = NCHUNK // NBLK
BLKE = BLKCH * CHUNK
ROWS_PER_TILE = N_NODES // NS


def _sc_body(ei_hbm, x_hbm, za_hbm, zc_hbm, outa_hbm, outc_hbm,
             src_a, src_b, dst_a, dst_b, src_t, dst_t, rows0, rows1, c_tile,
             a_sh, sem0, sem1, semi):
    c = lax.axis_index("c")
    s = lax.axis_index("s")
    wid = s * NC + c
    lo = s * ROWS_PER_TILE
    ebase = wid * EDGES_PER_W

    pltpu.sync_copy(za_hbm.at[pl.ds(lo, ROWS_PER_TILE)],
                    a_sh.at[pl.ds(lo, ROWS_PER_TILE)])
    pltpu.sync_copy(zc_hbm, c_tile)
    pltpu.sync_copy(ei_hbm.at[0].at[pl.ds(ebase, BLKE)], src_a)
    pltpu.sync_copy(ei_hbm.at[1].at[pl.ds(ebase, BLKE)], dst_a)
    pltpu.sync_copy(ei_hbm.at[0].at[pl.ds(ebase + NCHUNK * CHUNK, TAIL)], src_t)
    pltpu.sync_copy(ei_hbm.at[1].at[pl.ds(ebase + NCHUNK * CHUNK, TAIL)], dst_t)
    plsc.subcore_barrier()

    ones16 = jnp.ones((16,), jnp.float32)

    def gather(sv, j, rows, sem):
        return pltpu.async_copy(
            x_hbm.at[sv.at[pl.ds(j * CHUNK, CHUNK)]], rows, sem)

    def drain(rows, sem):
        pltpu.make_async_copy(x_hbm.at[src_a.at[pl.ds(0, CHUNK)]], rows,
                              sem).wait()

    def counts(dv, j):
        for k in range(CHUNK // 16):
            idx = dv[pl.ds(j * CHUNK + k * 16, 16)]
            plsc.addupdate_scatter(c_tile, [idx], ones16)

    def scatter(dv, j, rows):
        pltpu.sync_copy(rows, a_sh.at[dv.at[pl.ds(j * CHUNK, CHUNK)]],
                        add=True)

    idx_bufs = (src_a, dst_a), (src_b, dst_b)
    for blk in range(NBLK):
        sv, dv = idx_bufs[blk % 2]
        nsv, ndv = idx_bufs[(blk + 1) % 2]
        if blk + 1 < NBLK:
            nbase = ebase + (blk + 1) * BLKE
            pltpu.async_copy(ei_hbm.at[0].at[pl.ds(nbase, BLKE)], nsv, semi)
            pltpu.async_copy(ei_hbm.at[1].at[pl.ds(nbase, BLKE)], ndv, semi)

        gather(sv, 0, rows0, sem0)

        def pair_body(i, carry, sv=sv, dv=dv):
            j = 2 * i
            gather(sv, j + 1, rows1, sem1)
            counts(dv, j)
            drain(rows0, sem0)
            scatter(dv, j, rows0)
            gather(sv, j + 2, rows0, sem0)
            counts(dv, j + 1)
            drain(rows1, sem1)
            scatter(dv, j + 1, rows1)
            return carry

        lax.fori_loop(0, BLKCH // 2 - 1, pair_body, 0)
        gather(sv, BLKCH - 1, rows1, sem1)
        counts(dv, BLKCH - 2)
        drain(rows0, sem0)
        scatter(dv, BLKCH - 2, rows0)
        counts(dv, BLKCH - 1)
        drain(rows1, sem1)
        scatter(dv, BLKCH - 1, rows1)

        if blk + 1 < NBLK:
            pltpu.make_async_copy(ei_hbm.at[0].at[pl.ds(0, BLKE)], nsv,
                                  semi).wait()
            pltpu.make_async_copy(ei_hbm.at[1].at[pl.ds(0, BLKE)], ndv,
                                  semi).wait()

    pltpu.async_copy(x_hbm.at[src_t], rows0.at[pl.ds(0, TAIL)], sem0).wait()
    plsc.addupdate_scatter(c_tile, [dst_t[...]], ones16)
    pltpu.sync_copy(rows0.at[pl.ds(0, TAIL)], a_sh.at[dst_t], add=True)
    plsc.subcore_barrier()

    pltpu.sync_copy(a_sh.at[pl.ds(lo, ROWS_PER_TILE)],
                    outa_hbm.at[c].at[pl.ds(lo, ROWS_PER_TILE)])
    pltpu.sync_copy(c_tile, outc_hbm.at[c].at[s])


def _sc_aggregate(edge_index, x, zeros_a, zeros_c):
    mesh = plsc.VectorSubcoreMesh(core_axis_name="c", subcore_axis_name="s",
                                  num_cores=NC, num_subcores=NS)
    return pl.kernel(
        _sc_body,
        out_type=(
            jax.ShapeDtypeStruct((NC, N_NODES, D_FEAT), jnp.float32),
            jax.ShapeDtypeStruct((NC, NS, N_NODES), jnp.float32),
        ),
        mesh=mesh,
        compiler_params=pltpu.CompilerParams(use_tc_tiling_on_sc=False,
                                             needs_layout_passes=False),
        scratch_types=[
            pltpu.VMEM((BLKE,), jnp.int32),
            pltpu.VMEM((BLKE,), jnp.int32),
            pltpu.VMEM((BLKE,), jnp.int32),
            pltpu.VMEM((BLKE,), jnp.int32),
            pltpu.VMEM((TAIL,), jnp.int32),
            pltpu.VMEM((TAIL,), jnp.int32),
            pltpu.VMEM((CHUNK, D_FEAT), jnp.float32),
            pltpu.VMEM((CHUNK, D_FEAT), jnp.float32),
            pltpu.VMEM((N_NODES,), jnp.float32),
            pltpu.VMEM_SHARED((N_NODES, D_FEAT), jnp.float32),
            pltpu.SemaphoreType.DMA,
            pltpu.SemaphoreType.DMA,
            pltpu.SemaphoreType.DMA,
        ],
    )(edge_index, x, zeros_a, zeros_c)


def _mm_body(a_ref, c_ref, w_ref, b_ref, o_ref):
    a = a_ref[0] + a_ref[1]
    cnt = jnp.sum(c_ref[...].reshape(NC * NS, N_NODES), axis=0)
    o_ref[...] = (jnp.dot(a, w_ref[...], preferred_element_type=jnp.float32)
                  + cnt[:, None] * b_ref[...])


def _tc_finish(parts_a, parts_c, W, b2d):
    return pl.pallas_call(
        _mm_body,
        out_shape=jax.ShapeDtypeStruct((N_NODES, D_FEAT), jnp.float32),
    )(parts_a, parts_c, W, b2d)


def kernel(x, edge_index, W, b):
    ei = edge_index.astype(jnp.int32)
    zeros_a = jnp.zeros((N_NODES, D_FEAT), jnp.float32)
    zeros_c = jnp.zeros((N_NODES,), jnp.float32)
    parts_a, parts_c = _sc_aggregate(ei, x, zeros_a, zeros_c)
    return _tc_finish(parts_a, parts_c, W, b.reshape(1, D_FEAT))
